# baseline (speedup 1.0000x reference)
import jax
import jax.numpy as jnp
from jax import lax
from jax.experimental import pallas as pl
from jax.experimental.pallas import tpu as pltpu
from jax.experimental.pallas import tpu_sc as plsc

BATCH = 4096
EMB = 32
_NC = 2
_NS = 16
_NW = _NC * _NS
_BPW = BATCH // _NW
_CHUNK = 8
_IDXPAD = _BPW + 32


def _gather_one(tab_t, dummy, idx_v, ring, obuf, sem):
    nchunk = _BPW // _CHUNK
    cidx = jnp.arange(16, dtype=jnp.int32) % _CHUNK

    def fire(k, buf):
        vec = idx_v[pl.ds(k * _CHUNK, 16)]
        for l in range(_CHUNK):
            uid = vec[l]
            col = pl.multiple_of((uid // 128) * 128, 128)
            pltpu.async_copy(tab_t.at[:, pl.ds(col, 128)],
                             ring.at[buf, l], sem)

    def drain_extract(k, buf):
        pltpu.make_async_copy(dummy, ring.at[buf], sem).wait()
        vec = idx_v[pl.ds(k * _CHUNK, 16)]
        lanes = lax.gather(
            vec, cidx[:, None],
            dimension_numbers=lax.GatherDimensionNumbers(
                offset_dims=(), collapsed_slice_dims=(0,),
                start_index_map=(0,)),
            slice_sizes=(1,),
            mode=lax.GatherScatterMode.PROMISE_IN_BOUNDS) % 128
        jcol = k * _CHUNK + cidx
        dhalf = jnp.arange(16, dtype=jnp.int32) // _CHUNK
        for dp in range(EMB // 2):
            dvec = 2 * dp + dhalf
            vals = plsc.load_gather(ring.at[buf], [cidx, dvec, lanes])
            plsc.store_scatter(obuf, [dvec, jcol], vals)

    fire(0, 0)
    fire(1, 1)
    fire(2, 2)

    def triple(p, carry):
        for q in range(3):
            k = 3 * p + q

            @pl.when(k < nchunk)
            def _(k=k, q=q):
                drain_extract(k, q)

                @pl.when(k + 3 < nchunk)
                def _():
                    fire(k + 3, q)

        return carry

    lax.fori_loop(0, (nchunk + 2) // 3, triple, 0)


def _sc_gather_body(uids_hbm, iids_hbm, utab_t, itab_t, dummy, ut_out, it_out,
                    uidx_v, iidx_v, ring, ubuf, ibuf, sem):
    wid = lax.axis_index("s") * _NC + lax.axis_index("c")
    base = wid * _BPW
    pltpu.sync_copy(uids_hbm.at[pl.ds(base, _BPW)], uidx_v.at[pl.ds(0, _BPW)])
    pltpu.sync_copy(iids_hbm.at[pl.ds(base, _BPW)], iidx_v.at[pl.ds(0, _BPW)])
    _gather_one(utab_t, dummy, uidx_v, ring, ubuf, sem)
    _gather_one(itab_t, dummy, iidx_v, ring, ibuf, sem)
    pltpu.sync_copy(ubuf, ut_out.at[:, pl.ds(base, _BPW)])
    pltpu.sync_copy(ibuf, it_out.at[:, pl.ds(base, _BPW)])


def _sc_gather(user_ids, item_ids, utab_t, itab_t):
    mesh = plsc.VectorSubcoreMesh(core_axis_name="c", subcore_axis_name="s")
    kfn = pl.kernel(
        _sc_gather_body,
        mesh=mesh,
        out_type=[
            jax.ShapeDtypeStruct((EMB, BATCH), jnp.float32),
            jax.ShapeDtypeStruct((EMB, BATCH), jnp.float32),
        ],
        scratch_types=[
            pltpu.VMEM((_IDXPAD,), jnp.int32),
            pltpu.VMEM((_IDXPAD,), jnp.int32),
            pltpu.VMEM((3, _CHUNK, EMB, 128), jnp.float32),
            pltpu.VMEM((EMB, _BPW), jnp.float32),
            pltpu.VMEM((EMB, _BPW), jnp.float32),
            pltpu.SemaphoreType.DMA,
        ],
        compiler_params=pltpu.CompilerParams(use_tc_tiling_on_sc=True,
                                             needs_layout_passes=False),
    )
    dummy = jnp.zeros((_CHUNK, EMB, 128), jnp.float32)
    return kfn(user_ids, item_ids, utab_t, itab_t, dummy)


def _tc_dense_body(ut_ref, it_ref, w1_ref, b1_ref, w2_ref, b2_ref,
                   pred_ref, score_ref):
    ut = ut_ref[...]
    it = it_ref[...]
    s = jnp.sum(ut, axis=1, keepdims=True)
    pred_ref[...] = jnp.sum(it * s, axis=0, keepdims=True)
    uit = ut * it
    w1 = w1_ref[...]
    h = (jnp.dot(w1[:, :EMB], ut, preferred_element_type=jnp.float32)
         + jnp.dot(w1[:, EMB:2 * EMB], it, preferred_element_type=jnp.float32)
         + jnp.dot(w1[:, 2 * EMB:], uit, preferred_element_type=jnp.float32)
         + b1_ref[...].reshape(64, 1))
    h = jnp.maximum(h, 0.0)
    score_ref[...] = (jnp.sum(h * w2_ref[...].reshape(64, 1), axis=0,
                              keepdims=True) + b2_ref[...].reshape(1, 1))


def _tc_dense(ut, it, W1, b1, W2, b2):
    return pl.pallas_call(
        _tc_dense_body,
        out_shape=[
            jax.ShapeDtypeStruct((1, BATCH), jnp.float32),
            jax.ShapeDtypeStruct((1, BATCH), jnp.float32),
        ],
    )(ut, it, W1, b1, W2, b2)


def kernel(user_ids, item_ids, user_table, item_table, W1, b1, W2, b2):
    uids = user_ids.astype(jnp.int32)
    iids = item_ids.astype(jnp.int32)
    ut, it = _sc_gather(uids, iids, user_table.T, item_table.T)
    pred, score = _tc_dense(ut, it, W1, b1, W2, b2)
    return (pred[0], score[0])
